# baseline (speedup 1.0000x reference)
import jax
import jax.numpy as jnp
from jax.experimental import pallas as pl
from jax.experimental.pallas import tpu as pltpu

N = 32768
D = 512
C = 10
S = 16
BN = 8192
GRID = N // BN


def _fused(emb_ref, bi_ref, w_ref, b_ref, qtot_ref, qt_ref,
           hact_s, oh_s, sums_s):
    i = pl.program_id(0)
    h = jax.lax.dot_general(
        w_ref[...], emb_ref[...], (((0,), (1,)), ((), ())),
        preferred_element_type=jnp.float32) + b_ref[...]
    row = jax.lax.broadcasted_iota(jnp.int32, (2 * C, BN), 0)
    hact = jnp.where(row < C, jax.nn.softplus(h), h)
    hact_s[:, pl.ds(i * BN, BN)] = hact

    bi = bi_ref[...]
    oh = (bi == jax.lax.broadcasted_iota(jnp.int32, (S, BN), 0)
          ).astype(jnp.float32)
    oh_s[:, pl.ds(i * BN, BN)] = oh
    part = jax.lax.dot_general(
        oh, hact, (((1,), (1,)), ((), ())),
        preferred_element_type=jnp.float32)

    @pl.when(i == 0)
    def _init():
        sums_s[...] = part

    @pl.when(i != 0)
    def _acc():
        sums_s[...] += part

    @pl.when(i == GRID - 1)
    def _finale():
        sums = sums_s[...]
        wsum = sums[:, :C]
        qsum = sums[:, C:]
        dq = qtot_ref[...] - qsum
        fsys = jnp.where(wsum > 0, dq / jnp.where(wsum > 0, wsum, 1.0), 0.0)
        f = jax.lax.dot_general(
            fsys, oh_s[...], (((0,), (0,)), ((), ())),
            preferred_element_type=jnp.float32)
        hall = hact_s[...]
        qt_ref[...] = hall[C:, :] + hall[:C, :] * f


@jax.jit
def _run(embedding, batch_index, total_charge, W_wi, b_wi, W_qi, b_qi):
    bi_row = batch_index.reshape(1, N)
    w_cat = jnp.concatenate([W_wi, W_qi], axis=1)
    b_cat = jnp.concatenate([b_wi, b_qi]).reshape(2 * C, 1)
    qtot = total_charge.reshape(S, 1)

    q_t = pl.pallas_call(
        _fused,
        grid=(GRID,),
        in_specs=[
            pl.BlockSpec((BN, D), lambda i: (i, 0)),
            pl.BlockSpec((1, BN), lambda i: (0, i)),
            pl.BlockSpec((D, 2 * C), lambda i: (0, 0)),
            pl.BlockSpec((2 * C, 1), lambda i: (0, 0)),
            pl.BlockSpec((S, 1), lambda i: (0, 0)),
        ],
        out_specs=pl.BlockSpec((C, N), lambda i: (0, 0)),
        out_shape=jax.ShapeDtypeStruct((C, N), jnp.float32),
        scratch_shapes=[
            pltpu.VMEM((2 * C, N), jnp.float32),
            pltpu.VMEM((S, N), jnp.float32),
            pltpu.VMEM((S, 2 * C), jnp.float32),
        ],
    )(embedding, bi_row, w_cat, b_cat, qtot)
    return q_t.T


def kernel(embedding, coordinates, batch_index, natoms, total_charge,
           W_wi, b_wi, W_qi, b_qi):
    del coordinates, natoms
    return _run(embedding.astype(jnp.float32), batch_index,
                total_charge.astype(jnp.float32), W_wi, b_wi, W_qi, b_qi)


# speedup vs baseline: 1.0539x; 1.0539x over previous
import jax
import jax.numpy as jnp
from jax.experimental import pallas as pl
from jax.experimental.pallas import tpu as pltpu

N = 32768
D = 512
C = 10
S = 16
BN = 4096
GRID = N // BN


def _fused(emb_ref, bi_ref, w_ref, b_ref, qtot_ref, qt_ref,
           hact_s, sums_s):
    i = pl.program_id(0)
    h = jax.lax.dot_general(
        w_ref[...], emb_ref[...], (((0,), (1,)), ((), ())),
        preferred_element_type=jnp.float32) + b_ref[...]
    row = jax.lax.broadcasted_iota(jnp.int32, (2 * C, BN), 0)
    hact = jnp.where(row < C, jax.nn.softplus(h), h)
    hact_s[:, pl.ds(i * BN, BN)] = hact

    bi = bi_ref[:, pl.ds(i * BN, BN)]
    oh = (bi == jax.lax.broadcasted_iota(jnp.int32, (S, BN), 0)
          ).astype(jnp.float32)
    part = jax.lax.dot_general(
        oh, hact, (((1,), (1,)), ((), ())),
        preferred_element_type=jnp.float32)

    @pl.when(i == 0)
    def _init():
        sums_s[...] = part

    @pl.when(i != 0)
    def _acc():
        sums_s[...] += part

    @pl.when(i == GRID - 1)
    def _finale():
        sums = sums_s[...]
        wsum = sums[:, :C]
        qsum = sums[:, C:]
        dq = qtot_ref[...] - qsum
        fsys = jnp.where(wsum > 0, dq / jnp.where(wsum > 0, wsum, 1.0), 0.0)
        ohall = (bi_ref[...] ==
                 jax.lax.broadcasted_iota(jnp.int32, (S, N), 0)
                 ).astype(jnp.float32)
        f = jax.lax.dot_general(
            fsys, ohall, (((0,), (0,)), ((), ())),
            preferred_element_type=jnp.float32)
        hall = hact_s[...]
        qt_ref[...] = hall[C:, :] + hall[:C, :] * f


@jax.jit
def _run(embedding, batch_index, total_charge, W_wi, b_wi, W_qi, b_qi):
    bi_row = batch_index.reshape(1, N)
    w_cat = jnp.concatenate([W_wi, W_qi], axis=1)
    b_cat = jnp.concatenate([b_wi, b_qi]).reshape(2 * C, 1)
    qtot = total_charge.reshape(S, 1)

    q_t = pl.pallas_call(
        _fused,
        grid=(GRID,),
        in_specs=[
            pl.BlockSpec((BN, D), lambda i: (i, 0)),
            pl.BlockSpec((1, N), lambda i: (0, 0)),
            pl.BlockSpec((D, 2 * C), lambda i: (0, 0)),
            pl.BlockSpec((2 * C, 1), lambda i: (0, 0)),
            pl.BlockSpec((S, 1), lambda i: (0, 0)),
        ],
        out_specs=pl.BlockSpec((C, N), lambda i: (0, 0)),
        out_shape=jax.ShapeDtypeStruct((C, N), jnp.float32),
        scratch_shapes=[
            pltpu.VMEM((2 * C, N), jnp.float32),
            pltpu.VMEM((S, 2 * C), jnp.float32),
        ],
    )(embedding, bi_row, w_cat, b_cat, qtot)
    return q_t.T


def kernel(embedding, coordinates, batch_index, natoms, total_charge,
           W_wi, b_wi, W_qi, b_qi):
    del coordinates, natoms
    return _run(embedding.astype(jnp.float32), batch_index,
                total_charge.astype(jnp.float32), W_wi, b_wi, W_qi, b_qi)
